# baseline (speedup 1.0000x reference)
import functools

import jax
import jax.numpy as jnp
from jax import lax
from jax.experimental import pallas as pl
from jax.experimental.pallas import tpu as pltpu
from jax.experimental.pallas import tpu_sc as plsc

B, T, D = 16, 2048, 768
N = 800
MAX_BBOX = 100
L = 16
NC, NS = 2, 16
NW = NC * NS

B_SC = 8
B_TC = B - B_SC
W = NW // B_SC
R = T // W
C = 64
NCH = R // C

BCHUNK = 2
NB = B_TC // BCHUNK


def _sc_mean_body(x_hbm, out_hbm, buf0, buf1, acc, sem0, sem1):
    wid = lax.axis_index("s") * NC + lax.axis_index("c")
    img_off = wid // W
    sl = wid % W
    img = B_TC + img_off
    t0 = sl * R

    for j in range(D // L):
        acc[pl.ds(j * L, L)] = jnp.zeros((L,), jnp.float32)

    pltpu.async_copy(x_hbm.at[img, pl.ds(t0, C)], buf0, sem0)
    for c in range(NCH):
        buf, sem = (buf0, sem0) if c % 2 == 0 else (buf1, sem1)
        nbuf, nsem = (buf1, sem1) if c % 2 == 0 else (buf0, sem0)
        if c + 1 < NCH:
            pltpu.async_copy(x_hbm.at[img, pl.ds(t0 + (c + 1) * C, C)],
                             nbuf, nsem)
        pltpu.make_async_copy(x_hbm.at[img, pl.ds(t0, C)], buf, sem).wait()

        def row_body(r, _, buf=buf):
            for j in range(D // L):
                plsc.addupdate(acc.at[pl.ds(j * L, L)],
                               buf[r, pl.ds(j * L, L)])
            return 0

        lax.fori_loop(0, C, row_body, 0)

    for j in range(D // L):
        acc[pl.ds(j * L, L)] = acc[pl.ds(j * L, L)] * (1.0 / T)
    pltpu.sync_copy(acc, out_hbm.at[sl * B_SC + img_off])


_sc_mean = functools.partial(
    pl.kernel,
    out_type=jax.ShapeDtypeStruct((NW, D), jnp.float32),
    mesh=plsc.VectorSubcoreMesh(core_axis_name="c", subcore_axis_name="s"),
    scratch_types=[
        pltpu.VMEM((C, D), jnp.float32),
        pltpu.VMEM((C, D), jnp.float32),
        pltpu.VMEM((D,), jnp.float32),
        pltpu.SemaphoreType.DMA,
        pltpu.SemaphoreType.DMA,
    ],
)(_sc_mean_body)


def _tc_body(x_ref, ids_col_ref, ids_row_ref, feat_ref, w1_ref, b1_ref,
             w2_ref, b2_ref, pw_ref, pb_ref, vis_ref, mask_ref, retx_ref):
    b = pl.program_id(0)
    x = x_ref[...]
    retx_ref[...] = jnp.sum(x, axis=1, keepdims=True) * (1.0 / T)

    @pl.when(b == 0)
    def _boxes():
        ids_col = ids_col_ref[...]
        ids_row = ids_row_ref[...]
        f = feat_ref[...]
        h = jnp.maximum(
            jnp.dot(f, w1_ref[...], preferred_element_type=jnp.float32)
            + b1_ref[...], 0.0)
        f2 = (jnp.dot(h, w2_ref[...], preferred_element_type=jnp.float32)
              + b2_ref[...])
        xp = (jnp.dot(f2, pw_ref[...], preferred_element_type=jnp.float32)
              + pb_ref[...])

        eq = (ids_col == ids_row)
        ri = lax.broadcasted_iota(jnp.int32, (N, N), 0)
        ci = lax.broadcasted_iota(jnp.int32, (N, N), 1)
        before = jnp.logical_and(eq, ri < ci).astype(jnp.int32)
        slot_row = jnp.sum(before, axis=0, keepdims=True)
        pos_row = ids_row * MAX_BBOX + slot_row

        pp = lax.broadcasted_iota(jnp.int32, (B * MAX_BBOX, N), 0)
        P = (pp == pos_row).astype(jnp.float32)
        vis_ref[...] = jnp.dot(P, xp, preferred_element_type=jnp.float32)

        img = lax.broadcasted_iota(jnp.int32, (B, N), 0)
        counts = jnp.sum((img == ids_row).astype(jnp.int32), axis=1,
                         keepdims=True)
        jj = lax.broadcasted_iota(jnp.int32, (B, MAX_BBOX), 1)
        mask_ref[...] = (jj < counts).astype(jnp.float32)


def _combine_body(tc_ref, sc_ref, out_ref):
    p = sc_ref[...]
    s = p[0 * B_SC:1 * B_SC]
    for w in range(1, W):
        s = s + p[w * B_SC:(w + 1) * B_SC]
    out_ref[0:B_TC, :] = tc_ref[...]
    out_ref[B_TC:B, :] = s


def kernel(inputs, bboxes, features, mlp_w1, mlp_b1, mlp_w2, mlp_b2,
           tr_w1, tr_b1, tr_w2, tr_b2, proj_w, proj_b):
    del tr_w1, tr_b1, tr_w2, tr_b2
    ids = bboxes[:, 0]
    ids_col = ids.reshape(N, 1)
    ids_row = ids.reshape(1, N)
    feat = features[:, 1:]
    pw = proj_w[768:]

    sc_partials = _sc_mean(inputs)

    vis_flat, att_mask, retx_tc = pl.pallas_call(
        _tc_body,
        grid=(NB,),
        in_specs=[
            pl.BlockSpec((BCHUNK, T, D), lambda b: (b, 0, 0)),
            pl.BlockSpec((N, 1), lambda b: (0, 0)),
            pl.BlockSpec((1, N), lambda b: (0, 0)),
            pl.BlockSpec((N, 256), lambda b: (0, 0)),
            pl.BlockSpec((256, D), lambda b: (0, 0)),
            pl.BlockSpec((1, D), lambda b: (0, 0)),
            pl.BlockSpec((D, D), lambda b: (0, 0)),
            pl.BlockSpec((1, D), lambda b: (0, 0)),
            pl.BlockSpec((D, D), lambda b: (0, 0)),
            pl.BlockSpec((1, D), lambda b: (0, 0)),
        ],
        out_specs=[
            pl.BlockSpec((B * MAX_BBOX, D), lambda b: (0, 0)),
            pl.BlockSpec((B, MAX_BBOX), lambda b: (0, 0)),
            pl.BlockSpec((BCHUNK, 1, D), lambda b: (b, 0, 0)),
        ],
        out_shape=[
            jax.ShapeDtypeStruct((B * MAX_BBOX, D), jnp.float32),
            jax.ShapeDtypeStruct((B, MAX_BBOX), jnp.float32),
            jax.ShapeDtypeStruct((B_TC, 1, D), jnp.float32),
        ],
    )(inputs, ids_col, ids_row, feat, mlp_w1, mlp_b1.reshape(1, D),
      mlp_w2, mlp_b2.reshape(1, D), pw, proj_b.reshape(1, D))

    ret_x = pl.pallas_call(
        _combine_body,
        out_shape=jax.ShapeDtypeStruct((B, D), jnp.float32),
    )(retx_tc.reshape(B_TC, D), sc_partials)

    return (vis_flat.reshape(B, MAX_BBOX, D), att_mask, ret_x)


# speedup vs baseline: 2.0286x; 2.0286x over previous
import jax
import jax.numpy as jnp
from jax import lax
from jax.experimental import pallas as pl

B, T, D = 16, 2048, 768
N = 800
MAX_BBOX = 100


BCHUNK = 2
NB = B // BCHUNK


def _body(x_ref, ids_col_ref, ids_row_ref, feat_ref, w1_ref, b1_ref,
          w2_ref, b2_ref, pw_ref, pb_ref, vis_ref, mask_ref, retx_ref):
    b = pl.program_id(0)
    x = x_ref[...]
    retx_ref[...] = jnp.sum(x, axis=1, keepdims=True) * (1.0 / T)

    @pl.when(b == 0)
    def _boxes():
        ids_col = ids_col_ref[...]
        ids_row = ids_row_ref[...]
        f = feat_ref[...]
        h = jnp.maximum(
            jnp.dot(f, w1_ref[...], preferred_element_type=jnp.float32)
            + b1_ref[...], 0.0)
        f2 = (jnp.dot(h.astype(jnp.bfloat16), w2_ref[...],
                      preferred_element_type=jnp.float32) + b2_ref[...])
        xp = (jnp.dot(f2.astype(jnp.bfloat16), pw_ref[...],
                      preferred_element_type=jnp.float32)
              + pb_ref[...])

        eq = (ids_col == ids_row)
        ri = lax.broadcasted_iota(jnp.int32, (N, N), 0)
        ci = lax.broadcasted_iota(jnp.int32, (N, N), 1)
        before = jnp.logical_and(eq, ri < ci).astype(jnp.int32)
        slot_row = jnp.sum(before, axis=0, keepdims=True)
        pos_row = ids_row * MAX_BBOX + slot_row

        pp = lax.broadcasted_iota(jnp.int32, (B * MAX_BBOX, N), 0)
        P = (pp == pos_row).astype(jnp.float32)
        vis_ref[...] = jnp.dot(P, xp, preferred_element_type=jnp.float32)

        img = lax.broadcasted_iota(jnp.int32, (B, N), 0)
        counts = jnp.sum((img == ids_row).astype(jnp.int32), axis=1,
                         keepdims=True)
        jj = lax.broadcasted_iota(jnp.int32, (B, MAX_BBOX), 1)
        mask_ref[...] = (jj < counts).astype(jnp.float32)


def kernel(inputs, bboxes, features, mlp_w1, mlp_b1, mlp_w2, mlp_b2,
           tr_w1, tr_b1, tr_w2, tr_b2, proj_w, proj_b):
    del tr_w1, tr_b1, tr_w2, tr_b2
    ids = bboxes[:, 0]
    ids_col = ids.reshape(N, 1)
    ids_row = ids.reshape(1, N)
    feat = features[:, 1:]
    pw = proj_w[768:]

    vis_flat, att_mask, ret_x = pl.pallas_call(
        _body,
        grid=(NB,),
        in_specs=[
            pl.BlockSpec((BCHUNK, T, D), lambda b: (b, 0, 0)),
            pl.BlockSpec((N, 1), lambda b: (0, 0)),
            pl.BlockSpec((1, N), lambda b: (0, 0)),
            pl.BlockSpec((N, 256), lambda b: (0, 0)),
            pl.BlockSpec((256, D), lambda b: (0, 0)),
            pl.BlockSpec((1, D), lambda b: (0, 0)),
            pl.BlockSpec((D, D), lambda b: (0, 0)),
            pl.BlockSpec((1, D), lambda b: (0, 0)),
            pl.BlockSpec((D, D), lambda b: (0, 0)),
            pl.BlockSpec((1, D), lambda b: (0, 0)),
        ],
        out_specs=[
            pl.BlockSpec((B * MAX_BBOX, D), lambda b: (0, 0)),
            pl.BlockSpec((B, MAX_BBOX), lambda b: (0, 0)),
            pl.BlockSpec((BCHUNK, 1, D), lambda b: (b, 0, 0)),
        ],
        out_shape=[
            jax.ShapeDtypeStruct((B * MAX_BBOX, D), jnp.float32),
            jax.ShapeDtypeStruct((B, MAX_BBOX), jnp.float32),
            jax.ShapeDtypeStruct((B, 1, D), jnp.float32),
        ],
    )(inputs, ids_col, ids_row, feat.astype(jnp.bfloat16),
      mlp_w1.astype(jnp.bfloat16), mlp_b1.reshape(1, D),
      mlp_w2.astype(jnp.bfloat16), mlp_b2.reshape(1, D),
      pw.astype(jnp.bfloat16), proj_b.reshape(1, D))

    return (vis_flat.reshape(B, MAX_BBOX, D), att_mask, ret_x.reshape(B, D))


# speedup vs baseline: 2.0862x; 1.0284x over previous
import jax
import jax.numpy as jnp
from jax import lax
from jax.experimental import pallas as pl

B, T, D = 16, 2048, 768
N = 800
MAX_BBOX = 100


HB = B // 2


def _body(xa_ref, xb_ref, ids_col_ref, ids_row_ref, feat_ref, w1_ref, b1_ref,
          w2_ref, b2_ref, pw_ref, pb_ref, vis_ref, mask_ref,
          retxa_ref, retxb_ref):
    b = pl.program_id(0)
    retxa_ref[...] = jnp.sum(xa_ref[...], axis=1, keepdims=True) * (1.0 / T)
    retxb_ref[...] = jnp.sum(xb_ref[...], axis=1, keepdims=True) * (1.0 / T)

    @pl.when(b == 0)
    def _boxes():
        ids_col = ids_col_ref[...]
        ids_row = ids_row_ref[...]
        f = feat_ref[...]
        h = jnp.maximum(
            jnp.dot(f, w1_ref[...], preferred_element_type=jnp.float32)
            + b1_ref[...], 0.0)
        f2 = (jnp.dot(h, w2_ref[...], preferred_element_type=jnp.float32)
              + b2_ref[...])
        xp = (jnp.dot(f2, pw_ref[...], preferred_element_type=jnp.float32)
              + pb_ref[...])

        eq = (ids_col == ids_row)
        ri = lax.broadcasted_iota(jnp.int32, (N, N), 0)
        ci = lax.broadcasted_iota(jnp.int32, (N, N), 1)
        before = jnp.logical_and(eq, ri < ci).astype(jnp.int32)
        slot_row = jnp.sum(before, axis=0, keepdims=True)
        pos_row = ids_row * MAX_BBOX + slot_row

        pp = lax.broadcasted_iota(jnp.int32, (B * MAX_BBOX, N), 0)
        P = (pp == pos_row).astype(jnp.float32)
        vis_ref[...] = jnp.dot(P, xp, preferred_element_type=jnp.float32)

        img = lax.broadcasted_iota(jnp.int32, (B, N), 0)
        counts = jnp.sum((img == ids_row).astype(jnp.int32), axis=1,
                         keepdims=True)
        jj = lax.broadcasted_iota(jnp.int32, (B, MAX_BBOX), 1)
        mask_ref[...] = (jj < counts).astype(jnp.float32)


def kernel(inputs, bboxes, features, mlp_w1, mlp_b1, mlp_w2, mlp_b2,
           tr_w1, tr_b1, tr_w2, tr_b2, proj_w, proj_b):
    del tr_w1, tr_b1, tr_w2, tr_b2
    ids = bboxes[:, 0]
    ids_col = ids.reshape(N, 1)
    ids_row = ids.reshape(1, N)
    feat = features[:, 1:]
    pw = proj_w[768:]

    vis_flat, att_mask, retxa, retxb = pl.pallas_call(
        _body,
        grid=(HB,),
        in_specs=[
            pl.BlockSpec((1, T, D), lambda b: (b, 0, 0)),
            pl.BlockSpec((1, T, D), lambda b: (b + HB, 0, 0)),
            pl.BlockSpec((N, 1), lambda b: (0, 0)),
            pl.BlockSpec((1, N), lambda b: (0, 0)),
            pl.BlockSpec((N, 256), lambda b: (0, 0)),
            pl.BlockSpec((256, D), lambda b: (0, 0)),
            pl.BlockSpec((1, D), lambda b: (0, 0)),
            pl.BlockSpec((D, D), lambda b: (0, 0)),
            pl.BlockSpec((1, D), lambda b: (0, 0)),
            pl.BlockSpec((D, D), lambda b: (0, 0)),
            pl.BlockSpec((1, D), lambda b: (0, 0)),
        ],
        out_specs=[
            pl.BlockSpec((B * MAX_BBOX, D), lambda b: (0, 0)),
            pl.BlockSpec((B, MAX_BBOX), lambda b: (0, 0)),
            pl.BlockSpec((1, 1, D), lambda b: (b, 0, 0)),
            pl.BlockSpec((1, 1, D), lambda b: (b, 0, 0)),
        ],
        out_shape=[
            jax.ShapeDtypeStruct((B * MAX_BBOX, D), jnp.float32),
            jax.ShapeDtypeStruct((B, MAX_BBOX), jnp.float32),
            jax.ShapeDtypeStruct((HB, 1, D), jnp.float32),
            jax.ShapeDtypeStruct((HB, 1, D), jnp.float32),
        ],
    )(inputs, inputs, ids_col, ids_row, feat, mlp_w1, mlp_b1.reshape(1, D),
      mlp_w2, mlp_b2.reshape(1, D), pw, proj_b.reshape(1, D))

    ret_x = jnp.concatenate([retxa.reshape(HB, D), retxb.reshape(HB, D)])
    return (vis_flat.reshape(B, MAX_BBOX, D), att_mask, ret_x)


# speedup vs baseline: 2.1169x; 1.0147x over previous
import jax
import jax.numpy as jnp
from jax import lax
from jax.experimental import pallas as pl
from jax.experimental.pallas import tpu as pltpu

B, T, D = 16, 2048, 768
N = 800
MAX_BBOX = 100


BCHUNK = 2
NB = B // BCHUNK
VROWS = B * MAX_BBOX // NB


def _body(x_ref, ids_col_ref, ids_row_ref, feat_ref, w1_ref, b1_ref,
          w2_ref, b2_ref, pw_ref, pb_ref, vis_ref, mask_ref, retx_ref,
          xp_scr, pos_scr):
    b = pl.program_id(0)
    x = x_ref[...]
    retx_ref[...] = jnp.sum(x, axis=1, keepdims=True) * (1.0 / T)

    @pl.when(b == 0)
    def _boxes():
        ids_col = ids_col_ref[...]
        ids_row = ids_row_ref[...]
        f = feat_ref[...]
        h = jnp.maximum(
            jnp.dot(f, w1_ref[...], preferred_element_type=jnp.float32)
            + b1_ref[...], 0.0)
        f2 = (jnp.dot(h, w2_ref[...], preferred_element_type=jnp.float32)
              + b2_ref[...])
        xp = (jnp.dot(f2, pw_ref[...], preferred_element_type=jnp.float32)
              + pb_ref[...])

        eq = (ids_col == ids_row)
        ri = lax.broadcasted_iota(jnp.int32, (N, N), 0)
        ci = lax.broadcasted_iota(jnp.int32, (N, N), 1)
        before = jnp.logical_and(eq, ri < ci).astype(jnp.int32)
        slot_row = jnp.sum(before, axis=0, keepdims=True)
        pos_row = ids_row * MAX_BBOX + slot_row

        xp_scr[...] = xp
        pos_scr[...] = pos_row

        img = lax.broadcasted_iota(jnp.int32, (B, N), 0)
        counts = jnp.sum((img == ids_row).astype(jnp.int32), axis=1,
                         keepdims=True)
        jj = lax.broadcasted_iota(jnp.int32, (B, MAX_BBOX), 1)
        mask_ref[...] = (jj < counts).astype(jnp.float32)

    pp = lax.broadcasted_iota(jnp.int32, (VROWS, N), 0) + b * VROWS
    P = (pp == pos_scr[...]).astype(jnp.float32)
    vis_ref[...] = jnp.dot(P, xp_scr[...], preferred_element_type=jnp.float32)


def kernel(inputs, bboxes, features, mlp_w1, mlp_b1, mlp_w2, mlp_b2,
           tr_w1, tr_b1, tr_w2, tr_b2, proj_w, proj_b):
    del tr_w1, tr_b1, tr_w2, tr_b2
    ids = bboxes[:, 0]
    ids_col = ids.reshape(N, 1)
    ids_row = ids.reshape(1, N)
    feat = features[:, 1:]
    pw = proj_w[768:]

    vis_flat, att_mask, ret_x = pl.pallas_call(
        _body,
        grid=(NB,),
        in_specs=[
            pl.BlockSpec((BCHUNK, T, D), lambda b: (b, 0, 0)),
            pl.BlockSpec((N, 1), lambda b: (0, 0)),
            pl.BlockSpec((1, N), lambda b: (0, 0)),
            pl.BlockSpec((N, 256), lambda b: (0, 0)),
            pl.BlockSpec((256, D), lambda b: (0, 0)),
            pl.BlockSpec((1, D), lambda b: (0, 0)),
            pl.BlockSpec((D, D), lambda b: (0, 0)),
            pl.BlockSpec((1, D), lambda b: (0, 0)),
            pl.BlockSpec((D, D), lambda b: (0, 0)),
            pl.BlockSpec((1, D), lambda b: (0, 0)),
        ],
        out_specs=[
            pl.BlockSpec((VROWS, D), lambda b: (b, 0)),
            pl.BlockSpec((B, MAX_BBOX), lambda b: (0, 0)),
            pl.BlockSpec((BCHUNK, 1, D), lambda b: (b, 0, 0)),
        ],
        out_shape=[
            jax.ShapeDtypeStruct((B * MAX_BBOX, D), jnp.float32),
            jax.ShapeDtypeStruct((B, MAX_BBOX), jnp.float32),
            jax.ShapeDtypeStruct((B, 1, D), jnp.float32),
        ],
        scratch_shapes=[
            pltpu.VMEM((N, D), jnp.float32),
            pltpu.VMEM((1, N), jnp.int32),
        ],
    )(inputs, ids_col, ids_row, feat, mlp_w1, mlp_b1.reshape(1, D),
      mlp_w2, mlp_b2.reshape(1, D), pw, proj_b.reshape(1, D))

    return (vis_flat.reshape(B, MAX_BBOX, D), att_mask, ret_x.reshape(B, D))


# speedup vs baseline: 2.2267x; 1.0519x over previous
import jax
import jax.numpy as jnp
from jax import lax
from jax.experimental import pallas as pl
from jax.experimental.pallas import tpu as pltpu

B, T, D = 16, 2048, 768
N = 800
MAX_BBOX = 100


BCHUNK = 2
NB = B // BCHUNK
VROWS = B * MAX_BBOX // NB


def _body(x_ref, ids_col_ref, ids_row_ref, feat_ref, w1_ref, b1_ref,
          w2_ref, b2_ref, pw_ref, pb_ref, vis_ref, mask_ref, retx_ref,
          xp_scr, pos_scr):
    b = pl.program_id(0)
    x = x_ref[...]
    retx_ref[...] = jnp.sum(x, axis=1, keepdims=True) * (1.0 / T)

    @pl.when(b == 0)
    def _boxes():
        ids_col = ids_col_ref[...]
        ids_row = ids_row_ref[...]
        f = feat_ref[...][:, 1:]
        h = jnp.maximum(
            jnp.dot(f, w1_ref[...], preferred_element_type=jnp.float32)
            + b1_ref[...], 0.0)
        f2 = (jnp.dot(h, w2_ref[...], preferred_element_type=jnp.float32)
              + b2_ref[...])
        xp = (jnp.dot(f2, pw_ref[...], preferred_element_type=jnp.float32)
              + pb_ref[...])

        eq = (ids_col == ids_row)
        ri = lax.broadcasted_iota(jnp.int32, (N, N), 0)
        ci = lax.broadcasted_iota(jnp.int32, (N, N), 1)
        before = jnp.logical_and(eq, ri < ci).astype(jnp.int32)
        slot_row = jnp.sum(before, axis=0, keepdims=True)
        pos_row = ids_row * MAX_BBOX + slot_row

        xp_scr[...] = xp
        pos_scr[...] = pos_row

        img = lax.broadcasted_iota(jnp.int32, (B, N), 0)
        counts = jnp.sum((img == ids_row).astype(jnp.int32), axis=1,
                         keepdims=True)
        jj = lax.broadcasted_iota(jnp.int32, (B, MAX_BBOX), 1)
        mask_ref[...] = (jj < counts).astype(jnp.float32)

    pp = lax.broadcasted_iota(jnp.int32, (VROWS, N), 0) + b * VROWS
    P = (pp == pos_scr[...]).astype(jnp.float32)
    vis_ref[...] = jnp.dot(P, xp_scr[...], preferred_element_type=jnp.float32)


def kernel(inputs, bboxes, features, mlp_w1, mlp_b1, mlp_w2, mlp_b2,
           tr_w1, tr_b1, tr_w2, tr_b2, proj_w, proj_b):
    del tr_w1, tr_b1, tr_w2, tr_b2
    ids = bboxes[:, 0]
    ids_col = ids.reshape(N, 1)
    ids_row = ids.reshape(1, N)

    vis_flat, att_mask, ret_x = pl.pallas_call(
        _body,
        grid=(NB,),
        in_specs=[
            pl.BlockSpec((BCHUNK, T, D), lambda b: (b, 0, 0)),
            pl.BlockSpec((N, 1), lambda b: (0, 0)),
            pl.BlockSpec((1, N), lambda b: (0, 0)),
            pl.BlockSpec((N, 257), lambda b: (0, 0)),
            pl.BlockSpec((256, D), lambda b: (0, 0)),
            pl.BlockSpec((1, D), lambda b: (0, 0)),
            pl.BlockSpec((D, D), lambda b: (0, 0)),
            pl.BlockSpec((1, D), lambda b: (0, 0)),
            pl.BlockSpec((D, D), lambda b: (1, 0)),
            pl.BlockSpec((1, D), lambda b: (0, 0)),
        ],
        out_specs=[
            pl.BlockSpec((VROWS, D), lambda b: (b, 0)),
            pl.BlockSpec((B, MAX_BBOX), lambda b: (0, 0)),
            pl.BlockSpec((BCHUNK, 1, D), lambda b: (b, 0, 0)),
        ],
        out_shape=[
            jax.ShapeDtypeStruct((B * MAX_BBOX, D), jnp.float32),
            jax.ShapeDtypeStruct((B, MAX_BBOX), jnp.float32),
            jax.ShapeDtypeStruct((B, 1, D), jnp.float32),
        ],
        scratch_shapes=[
            pltpu.VMEM((N, D), jnp.float32),
            pltpu.VMEM((1, N), jnp.int32),
        ],
    )(inputs, ids_col, ids_row, features, mlp_w1, mlp_b1.reshape(1, D),
      mlp_w2, mlp_b2.reshape(1, D), proj_w, proj_b.reshape(1, D))

    return (vis_flat.reshape(B, MAX_BBOX, D), att_mask, ret_x.reshape(B, D))


# speedup vs baseline: 2.2553x; 1.0128x over previous
import jax
import jax.numpy as jnp
from jax import lax
from jax.experimental import pallas as pl
from jax.experimental.pallas import tpu as pltpu

B, T, D = 16, 2048, 768
N = 800
MAX_BBOX = 100


BCHUNK = 2
NB = B // BCHUNK
NVCH = 4
VROWS = B * MAX_BBOX // NVCH


def _body(x_ref, ids_col_ref, ids_row_ref, feat_ref, w1_ref, b1_ref,
          w2_ref, b2_ref, pw_ref, pb_ref, vis_ref, mask_ref, retx_ref,
          xp_scr, pos_scr, w1_s, w2_s, pw_s, sem, retsem):
    b = pl.program_id(0)
    x = x_ref[...]
    retx_ref[...] = jnp.sum(x, axis=1, keepdims=True) * (1.0 / T)

    @pl.when(b == 0)
    def _fetch_weights():
        pltpu.make_async_copy(w1_ref, w1_s, sem).start()
        pltpu.make_async_copy(w2_ref, w2_s, sem).start()
        pltpu.make_async_copy(pw_ref.at[pl.ds(D, D)], pw_s, sem).start()

    @pl.when(b == 1)
    def _boxes():
        pltpu.make_async_copy(w1_ref, w1_s, sem).wait()
        pltpu.make_async_copy(w2_ref, w2_s, sem).wait()
        pltpu.make_async_copy(pw_ref.at[pl.ds(D, D)], pw_s, sem).wait()
        ids_col = ids_col_ref[...]
        ids_row = ids_row_ref[...]
        f = feat_ref[...][:, 1:]
        h = jnp.maximum(
            jnp.dot(f, w1_s[...], preferred_element_type=jnp.float32)
            + b1_ref[...], 0.0)
        f2 = (jnp.dot(h, w2_s[...], preferred_element_type=jnp.float32)
              + b2_ref[...])
        xp = (jnp.dot(f2, pw_s[...], preferred_element_type=jnp.float32)
              + pb_ref[...])

        eq = (ids_col == ids_row)
        ri = lax.broadcasted_iota(jnp.int32, (N, N), 0)
        ci = lax.broadcasted_iota(jnp.int32, (N, N), 1)
        before = jnp.logical_and(eq, ri < ci).astype(jnp.int32)
        slot_row = jnp.sum(before, axis=0, keepdims=True)
        pos_row = ids_row * MAX_BBOX + slot_row

        xp_scr[...] = xp
        pos_scr[...] = pos_row

        img = lax.broadcasted_iota(jnp.int32, (B, N), 0)
        counts = jnp.sum((img == ids_row).astype(jnp.int32), axis=1,
                         keepdims=True)
        jj = lax.broadcasted_iota(jnp.int32, (B, MAX_BBOX), 1)
        mask_ref[...] = (jj < counts).astype(jnp.float32)

    @pl.when(b >= NB - NVCH)
    def _vis():
        cb = jnp.maximum(b - (NB - NVCH), 0)
        pp = lax.broadcasted_iota(jnp.int32, (VROWS, N), 0) + cb * VROWS
        P = (pp == pos_scr[...]).astype(jnp.float32)
        vis_ref[...] = jnp.dot(P, xp_scr[...],
                               preferred_element_type=jnp.float32)


def kernel(inputs, bboxes, features, mlp_w1, mlp_b1, mlp_w2, mlp_b2,
           tr_w1, tr_b1, tr_w2, tr_b2, proj_w, proj_b):
    del tr_w1, tr_b1, tr_w2, tr_b2
    ids = bboxes[:, 0]
    ids_col = ids.reshape(N, 1)
    ids_row = ids.reshape(1, N)

    vis_flat, att_mask, ret_x = pl.pallas_call(
        _body,
        grid=(NB,),
        in_specs=[
            pl.BlockSpec((BCHUNK, T, D), lambda b: (b, 0, 0)),
            pl.BlockSpec((N, 1), lambda b: (0, 0)),
            pl.BlockSpec((1, N), lambda b: (0, 0)),
            pl.BlockSpec((N, 257), lambda b: (0, 0)),
            pl.BlockSpec(memory_space=pltpu.MemorySpace.HBM),
            pl.BlockSpec((1, D), lambda b: (0, 0)),
            pl.BlockSpec(memory_space=pltpu.MemorySpace.HBM),
            pl.BlockSpec((1, D), lambda b: (0, 0)),
            pl.BlockSpec(memory_space=pltpu.MemorySpace.HBM),
            pl.BlockSpec((1, D), lambda b: (0, 0)),
        ],
        out_specs=[
            pl.BlockSpec((VROWS, D),
                         lambda b: (jnp.maximum(b - (NB - NVCH), 0), 0)),
            pl.BlockSpec((B, MAX_BBOX), lambda b: (0, 0)),
            pl.BlockSpec((BCHUNK, 1, D), lambda b: (b, 0, 0)),
        ],
        out_shape=[
            jax.ShapeDtypeStruct((B * MAX_BBOX, D), jnp.float32),
            jax.ShapeDtypeStruct((B, MAX_BBOX), jnp.float32),
            jax.ShapeDtypeStruct((B, 1, D), jnp.float32),
        ],
        scratch_shapes=[
            pltpu.VMEM((N, D), jnp.float32),
            pltpu.VMEM((1, N), jnp.int32),
            pltpu.VMEM((256, D), jnp.float32),
            pltpu.VMEM((D, D), jnp.float32),
            pltpu.VMEM((D, D), jnp.float32),
            pltpu.SemaphoreType.DMA,
            pltpu.SemaphoreType.DMA,
        ],
    )(inputs, ids_col, ids_row, features, mlp_w1, mlp_b1.reshape(1, D),
      mlp_w2, mlp_b2.reshape(1, D), proj_w, proj_b.reshape(1, D))

    return (vis_flat.reshape(B, MAX_BBOX, D), att_mask, ret_x.reshape(B, D))


# speedup vs baseline: 2.2647x; 1.0042x over previous
import jax
import jax.numpy as jnp
from jax import lax
from jax.experimental import pallas as pl
from jax.experimental.pallas import tpu as pltpu

B, T, D = 16, 2048, 768
N = 800
MAX_BBOX = 100


BCHUNK = 2
NB = B // BCHUNK
NVCH = 4
VROWS = B * MAX_BBOX // NVCH


def _body(x_ref, ids_col_ref, ids_row_ref, feat_ref, w1_ref, b1_ref,
          w2_ref, b2_ref, pw_ref, pb_ref, vis_ref, mask_ref, retx_ref,
          xp_scr, pos_scr, w1_s, w2_s, pw_s, f_s, sem):
    b = pl.program_id(0)
    x = x_ref[...]
    retx_ref[...] = jnp.sum(x, axis=1, keepdims=True) * (1.0 / T)

    @pl.when(b == 0)
    def _fetch_weights():
        pltpu.make_async_copy(w1_ref, w1_s, sem).start()
        pltpu.make_async_copy(w2_ref, w2_s, sem).start()
        pltpu.make_async_copy(pw_ref.at[pl.ds(D, D)], pw_s, sem).start()
        pltpu.make_async_copy(feat_ref, f_s, sem).start()

    @pl.when(b == 1)
    def _boxes():
        pltpu.make_async_copy(w1_ref, w1_s, sem).wait()
        pltpu.make_async_copy(w2_ref, w2_s, sem).wait()
        pltpu.make_async_copy(pw_ref.at[pl.ds(D, D)], pw_s, sem).wait()
        pltpu.make_async_copy(feat_ref, f_s, sem).wait()
        ids_col = ids_col_ref[...]
        ids_row = ids_row_ref[...]
        f = f_s[...][:, 1:]
        h = jnp.maximum(
            jnp.dot(f, w1_s[...], preferred_element_type=jnp.float32)
            + b1_ref[...], 0.0)
        f2 = (jnp.dot(h, w2_s[...], preferred_element_type=jnp.float32)
              + b2_ref[...])
        xp = (jnp.dot(f2, pw_s[...], preferred_element_type=jnp.float32)
              + pb_ref[...])

        eq = (ids_col == ids_row)
        ri = lax.broadcasted_iota(jnp.int32, (N, N), 0)
        ci = lax.broadcasted_iota(jnp.int32, (N, N), 1)
        before = jnp.logical_and(eq, ri < ci).astype(jnp.int32)
        slot_row = jnp.sum(before, axis=0, keepdims=True)
        pos_row = ids_row * MAX_BBOX + slot_row

        xp_scr[...] = xp
        pos_scr[...] = pos_row

        img = lax.broadcasted_iota(jnp.int32, (B, N), 0)
        counts = jnp.sum((img == ids_row).astype(jnp.int32), axis=1,
                         keepdims=True)
        jj = lax.broadcasted_iota(jnp.int32, (B, MAX_BBOX), 1)
        mask_ref[...] = (jj < counts).astype(jnp.float32)

    @pl.when(b >= NB - NVCH)
    def _vis():
        cb = jnp.maximum(b - (NB - NVCH), 0)
        pp = lax.broadcasted_iota(jnp.int32, (VROWS, N), 0) + cb * VROWS
        P = (pp == pos_scr[...]).astype(jnp.float32)
        vis_ref[...] = jnp.dot(P, xp_scr[...],
                               preferred_element_type=jnp.float32)


def kernel(inputs, bboxes, features, mlp_w1, mlp_b1, mlp_w2, mlp_b2,
           tr_w1, tr_b1, tr_w2, tr_b2, proj_w, proj_b):
    del tr_w1, tr_b1, tr_w2, tr_b2
    ids = bboxes[:, 0]
    ids_col = ids.reshape(N, 1)
    ids_row = ids.reshape(1, N)

    vis_flat, att_mask, ret_x = pl.pallas_call(
        _body,
        grid=(NB,),
        in_specs=[
            pl.BlockSpec((BCHUNK, T, D), lambda b: (b, 0, 0)),
            pl.BlockSpec((N, 1), lambda b: (0, 0)),
            pl.BlockSpec((1, N), lambda b: (0, 0)),
            pl.BlockSpec(memory_space=pltpu.MemorySpace.HBM),
            pl.BlockSpec(memory_space=pltpu.MemorySpace.HBM),
            pl.BlockSpec((1, D), lambda b: (0, 0)),
            pl.BlockSpec(memory_space=pltpu.MemorySpace.HBM),
            pl.BlockSpec((1, D), lambda b: (0, 0)),
            pl.BlockSpec(memory_space=pltpu.MemorySpace.HBM),
            pl.BlockSpec((1, D), lambda b: (0, 0)),
        ],
        out_specs=[
            pl.BlockSpec((VROWS, D),
                         lambda b: (jnp.maximum(b - (NB - NVCH), 0), 0)),
            pl.BlockSpec((B, MAX_BBOX), lambda b: (0, 0)),
            pl.BlockSpec((BCHUNK, 1, D), lambda b: (b, 0, 0)),
        ],
        out_shape=[
            jax.ShapeDtypeStruct((B * MAX_BBOX, D), jnp.float32),
            jax.ShapeDtypeStruct((B, MAX_BBOX), jnp.float32),
            jax.ShapeDtypeStruct((B, 1, D), jnp.float32),
        ],
        scratch_shapes=[
            pltpu.VMEM((N, D), jnp.float32),
            pltpu.VMEM((1, N), jnp.int32),
            pltpu.VMEM((256, D), jnp.float32),
            pltpu.VMEM((D, D), jnp.float32),
            pltpu.VMEM((D, D), jnp.float32),
            pltpu.VMEM((N, 257), jnp.float32),
            pltpu.SemaphoreType.DMA,
        ],
    )(inputs, ids_col, ids_row, features, mlp_w1, mlp_b1.reshape(1, D),
      mlp_w2, mlp_b2.reshape(1, D), proj_w, proj_b.reshape(1, D))

    return (vis_flat.reshape(B, MAX_BBOX, D), att_mask, ret_x.reshape(B, D))
